# baseline (speedup 1.0000x reference)
import functools

import jax
import jax.numpy as jnp
from jax import lax
from jax.experimental import pallas as pl
from jax.experimental.pallas import tpu as pltpu
from jax.experimental.pallas import tpu_sc as plsc

L = 16
NS = 16
N_IN = 16384
CPT = N_IN // NS
NV = 5
D = 14
VPI = 8

_mesh = plsc.VectorSubcoreMesh(core_axis_name="c", subcore_axis_name="s",
                               num_cores=1)


@functools.partial(
    pl.kernel,
    mesh=_mesh,
    out_type=jax.ShapeDtypeStruct((L,), jnp.float32),
    scratch_types=[
        pltpu.VMEM((CPT,), jnp.int32),
        pltpu.VMEM((L,), jnp.float32),
        pltpu.VMEM((NS, L), jnp.float32),
        pltpu.VMEM((5 * NV * L,), jnp.float32),
        pltpu.VMEM((10 * NV * L,), jnp.float32),
        pltpu.VMEM((5 * NV * L,), jnp.float32),
        pltpu.VMEM((D * 64,), jnp.float32),
        pltpu.VMEM((64,), jnp.float32),
        pltpu.VMEM_SHARED((NS, 128), jnp.float32),
    ],
)
def _sc_bag_chain(a_hbm, b_hbm, e_hbm, w0_hbm, w1_hbm, w2_hbm, out_hbm,
                  idx_v, part_v, red_v, w0_v, w1_v, w2_v, a_v, b_v, shared):
    s = lax.axis_index("s")
    lane = lax.broadcasted_iota(jnp.int32, (L,), 0)

    def lane_sum(x):
        for sh in (8, 4, 2, 1):
            x = x + x.at[lane ^ sh].get(mode="promise_in_bounds",
                                        unique_indices=True)
        return x

    def bcast(x, r):
        return x.at[jnp.full((L,), r, jnp.int32)].get(
            mode="promise_in_bounds")

    pltpu.sync_copy(e_hbm.at[pl.ds(s * CPT, CPT)], idx_v)

    @pl.when(s == 0)
    def _prefetch():
        pltpu.sync_copy(w0_hbm, w0_v)
        pltpu.sync_copy(w1_hbm, w1_v)
        pltpu.sync_copy(w2_hbm, w2_v)
        pltpu.sync_copy(a_hbm, a_v)
        pltpu.sync_copy(b_hbm, b_v)

    def body(it, acc):
        base = it * (VPI * L)
        for k in range(VPI):
            x = jnp.maximum(idx_v[pl.ds(base + k * L, L)], 0)
            acc = tuple(acc[v] + jnp.where(x == v, 1.0, 0.0)
                        for v in range(NV - 1))
        return acc

    acc = lax.fori_loop(0, CPT // (VPI * L), body,
                        tuple(jnp.zeros((L,), jnp.float32)
                              for _ in range(NV - 1)))
    cs = [lane_sum(a) for a in acc]
    cs.append(jnp.float32(CPT) - cs[0] - cs[1] - cs[2] - cs[3])
    part = jnp.zeros((L,), jnp.float32)
    for v in range(NV):
        part = jnp.where(lane == v, cs[v], part)
    part_v[...] = part
    pltpu.sync_copy(part_v, shared.at[s, pl.ds(0, L)])

    plsc.subcore_barrier()

    @pl.when(s == 0)
    def _tail():
        for i in range(NS):
            pltpu.sync_copy(shared.at[i, pl.ds(0, L)], part_v)
            red_v[i, :] = part_v[...]
        counts = jnp.zeros((L,), jnp.float32)
        for i in range(NS):
            counts = counts + red_v[i, :]
        cts = [bcast(counts, v) for v in range(NV)]

        def eb_sum(w_v, num_tables):
            e = jnp.zeros((L,), jnp.float32)
            for i in range(num_tables):
                for v in range(NV):
                    e = e + cts[v] * w_v[pl.ds((i * NV + v) * L, L)]
            return e

        e0 = eb_sum(w0_v, 5)
        e1 = eb_sum(w1_v, 10)
        e2 = eb_sum(w2_v, 5)

        def bf16_round(x):
            i = lax.bitcast_convert_type(x, jnp.int32)
            i = i + jnp.int32(0x7FFF) + jnp.bitwise_and(
                lax.shift_right_logical(i, 16), jnp.int32(1))
            i = jnp.bitwise_and(i, jnp.int32(-65536))
            return lax.bitcast_convert_type(i, jnp.float32)

        bfb = [bf16_round(b_v[pl.ds(q * L, L)]) for q in range(4)]
        mm0 = jnp.zeros((L,), jnp.float32)
        for r in range(D):
            prod = jnp.zeros((L,), jnp.float32)
            for q in range(4):
                prod = (prod
                        + bf16_round(a_v[pl.ds(r * 64 + q * L, L)]) * bfb[q])
            mm0 = jnp.where(lane == r, lane_sum(prod), mm0)

        be1 = bf16_round(e1)
        mm2 = jnp.zeros((L,), jnp.float32)
        for r in range(D):
            mm1_r = bcast(mm0, r) * e0
            mm2 = mm2 + bcast(be1, r) * bf16_round(mm1_r)

        out = lane_sum(e2 * mm2)
        part_v[...] = jnp.where(lane == 0, out, 0.0)
        pltpu.sync_copy(part_v, out_hbm)


def kernel(mm_0_a, mm_0_b, eb_input, eb_offset, W0, W1, W2):
    del eb_offset
    pad = ((0, 0), (0, 0), (0, L - D))
    w0p = jnp.pad(W0, pad).reshape(-1)
    w1p = jnp.pad(W1, pad).reshape(-1)
    w2p = jnp.pad(W2, pad).reshape(-1)
    out = _sc_bag_chain(mm_0_a.reshape(-1), mm_0_b.reshape(-1), eb_input,
                        w0p, w1p, w2p)
    return out[0:1].reshape(1, 1)


# speedup vs baseline: 1.0477x; 1.0477x over previous
import functools

import jax
import jax.numpy as jnp
from jax import lax
from jax.experimental import pallas as pl
from jax.experimental.pallas import tpu as pltpu
from jax.experimental.pallas import tpu_sc as plsc

L = 16
NS = 16
N_IN = 16384
CPT = N_IN // NS
NV = 5
D = 14
VPI = 8

_mesh = plsc.VectorSubcoreMesh(core_axis_name="c", subcore_axis_name="s",
                               num_cores=1)


@functools.partial(
    pl.kernel,
    mesh=_mesh,
    out_type=jax.ShapeDtypeStruct((L,), jnp.float32),
    scratch_types=[
        pltpu.VMEM((CPT,), jnp.int32),
        pltpu.VMEM((L,), jnp.float32),
        pltpu.VMEM((NS, 128), jnp.float32),
        pltpu.VMEM((5 * NV * L,), jnp.float32),
        pltpu.VMEM((10 * NV * L,), jnp.float32),
        pltpu.VMEM((5 * NV * L,), jnp.float32),
        pltpu.VMEM((D * 64,), jnp.float32),
        pltpu.VMEM((64,), jnp.float32),
        pltpu.VMEM_SHARED((NS, 128), jnp.float32),
    ],
)
def _sc_bag_chain(a_hbm, b_hbm, e_hbm, w0_hbm, w1_hbm, w2_hbm, out_hbm,
                  idx_v, part_v, red_v, w0_v, w1_v, w2_v, a_v, b_v, shared):
    s = lax.axis_index("s")
    lane = lax.broadcasted_iota(jnp.int32, (L,), 0)

    def lane_sum(x):
        for sh in (8, 4, 2, 1):
            x = x + x.at[lane ^ sh].get(mode="promise_in_bounds",
                                        unique_indices=True)
        return x

    def bcast(x, r):
        return x.at[jnp.full((L,), r, jnp.int32)].get(
            mode="promise_in_bounds")

    pltpu.sync_copy(e_hbm.at[pl.ds(s * CPT, CPT)], idx_v)

    @pl.when(s == 0)
    def _prefetch():
        pltpu.sync_copy(w0_hbm, w0_v)
        pltpu.sync_copy(w1_hbm, w1_v)
        pltpu.sync_copy(w2_hbm, w2_v)
        pltpu.sync_copy(a_hbm, a_v)
        pltpu.sync_copy(b_hbm, b_v)

    def body(it, acc):
        base = it * (VPI * L)
        for k in range(VPI):
            x = jnp.maximum(idx_v[pl.ds(base + k * L, L)], 0)
            acc = tuple(acc[v] + jnp.where(x == v, 1.0, 0.0)
                        for v in range(NV - 1))
        return acc

    acc = lax.fori_loop(0, CPT // (VPI * L), body,
                        tuple(jnp.zeros((L,), jnp.float32)
                              for _ in range(NV - 1)))
    cs = [lane_sum(a) for a in acc]
    cs.append(jnp.float32(CPT) - cs[0] - cs[1] - cs[2] - cs[3])
    part = jnp.zeros((L,), jnp.float32)
    for v in range(NV):
        part = jnp.where(lane == v, cs[v], part)
    part_v[...] = part
    pltpu.sync_copy(part_v, shared.at[s, pl.ds(0, L)])

    plsc.subcore_barrier()

    @pl.when(s == 0)
    def _tail():
        pltpu.sync_copy(shared, red_v)
        counts = jnp.zeros((L,), jnp.float32)
        for i in range(NS):
            counts = counts + red_v[i, pl.ds(0, L)]
        cts = [bcast(counts, v) for v in range(NV)]

        def eb_sum(w_v, num_tables):
            e = jnp.zeros((L,), jnp.float32)
            for i in range(num_tables):
                for v in range(NV):
                    e = e + cts[v] * w_v[pl.ds((i * NV + v) * L, L)]
            return e

        e0 = eb_sum(w0_v, 5)
        e1 = eb_sum(w1_v, 10)
        e2 = eb_sum(w2_v, 5)

        def bf16_round(x):
            i = lax.bitcast_convert_type(x, jnp.int32)
            i = i + jnp.int32(0x7FFF) + jnp.bitwise_and(
                lax.shift_right_logical(i, 16), jnp.int32(1))
            i = jnp.bitwise_and(i, jnp.int32(-65536))
            return lax.bitcast_convert_type(i, jnp.float32)

        bfb = [bf16_round(b_v[pl.ds(q * L, L)]) for q in range(4)]
        mm0 = jnp.zeros((L,), jnp.float32)
        for r in range(D):
            prod = jnp.zeros((L,), jnp.float32)
            for q in range(4):
                prod = (prod
                        + bf16_round(a_v[pl.ds(r * 64 + q * L, L)]) * bfb[q])
            mm0 = jnp.where(lane == r, lane_sum(prod), mm0)

        be1 = bf16_round(e1)
        mm2 = jnp.zeros((L,), jnp.float32)
        for r in range(D):
            mm1_r = bcast(mm0, r) * e0
            mm2 = mm2 + bcast(be1, r) * bf16_round(mm1_r)

        out = lane_sum(e2 * mm2)
        part_v[...] = jnp.where(lane == 0, out, 0.0)
        pltpu.sync_copy(part_v, out_hbm)


def kernel(mm_0_a, mm_0_b, eb_input, eb_offset, W0, W1, W2):
    del eb_offset
    pad = ((0, 0), (0, 0), (0, L - D))
    w0p = jnp.pad(W0, pad).reshape(-1)
    w1p = jnp.pad(W1, pad).reshape(-1)
    w2p = jnp.pad(W2, pad).reshape(-1)
    out = _sc_bag_chain(mm_0_a.reshape(-1), mm_0_b.reshape(-1), eb_input,
                        w0p, w1p, w2p)
    return out[0:1].reshape(1, 1)


# speedup vs baseline: 1.1526x; 1.1002x over previous
import functools

import jax
import jax.numpy as jnp
from jax import lax
from jax.experimental import pallas as pl
from jax.experimental.pallas import tpu as pltpu
from jax.experimental.pallas import tpu_sc as plsc

L = 16
NW = 16
N_IN = 16384
CPT = N_IN // NW
NV = 5
D = 14

_mesh = plsc.VectorSubcoreMesh(core_axis_name="c", subcore_axis_name="s",
                               num_cores=1)


@functools.partial(
    pl.kernel,
    mesh=_mesh,
    out_type=jax.ShapeDtypeStruct((NW, L), jnp.float32),
    scratch_types=[
        pltpu.VMEM((CPT,), jnp.int32),
        pltpu.VMEM((L,), jnp.float32),
    ],
)
def _sc_histogram(e_hbm, out_hbm, idx_v, part_v):
    s = lax.axis_index("s")
    wid = s
    lane = lax.broadcasted_iota(jnp.int32, (L,), 0)

    def lane_sum(x):
        for sh in (8, 4, 2, 1):
            x = x + x.at[lane ^ sh].get(mode="promise_in_bounds",
                                        unique_indices=True)
        return x

    pltpu.sync_copy(e_hbm.at[pl.ds(wid * CPT, CPT)], idx_v)

    VPI = 16

    def body(it, acc):
        base = it * (VPI * L)
        for k in range(VPI):
            x = jnp.maximum(idx_v[pl.ds(base + k * L, L)], 0)
            acc = tuple(acc[v] + jnp.where(x == v, 1.0, 0.0)
                        for v in range(NV - 1))
        return acc

    acc = lax.fori_loop(0, CPT // (VPI * L), body,
                        tuple(jnp.zeros((L,), jnp.float32)
                              for _ in range(NV - 1)))
    cs = [lane_sum(a) for a in acc]
    cs.append(jnp.float32(CPT) - cs[0] - cs[1] - cs[2] - cs[3])
    part = jnp.zeros((L,), jnp.float32)
    for v in range(NV):
        part = jnp.where(lane == v, cs[v], part)
    part_v[...] = part
    pltpu.sync_copy(part_v, out_hbm.at[wid])


def _tc_tail(part_ref, a_ref, b_ref, w0_ref, w1_ref, w2_ref, out_ref):
    bf = jnp.bfloat16
    counts = jnp.sum(part_ref[...], axis=0)
    c5 = counts[:NV]
    e0 = jnp.sum(jnp.sum(w0_ref[...], axis=0) * c5[:, None], axis=0)
    e1 = jnp.sum(jnp.sum(w1_ref[...], axis=0) * c5[:, None], axis=0)
    e2 = jnp.sum(jnp.sum(w2_ref[...], axis=0) * c5[:, None], axis=0)
    mm0 = jax.lax.dot_general(
        a_ref[...].astype(bf), b_ref[...].astype(bf),
        (((1,), (0,)), ((), ())),
        preferred_element_type=jnp.float32)
    mm1 = mm0 * e0[None, :]
    mm2 = jax.lax.dot_general(
        e1[None, :].astype(bf), mm1.astype(bf),
        (((1,), (0,)), ((), ())),
        preferred_element_type=jnp.float32)
    s = jnp.sum(e2 * mm2[0, :])
    out_ref[...] = jnp.full((1, 1), s, jnp.float32)


_tc_tail_call = pl.pallas_call(
    _tc_tail,
    out_shape=jax.ShapeDtypeStruct((1, 1), jnp.float32),
)


def kernel(mm_0_a, mm_0_b, eb_input, eb_offset, W0, W1, W2):
    del eb_offset
    part = _sc_histogram(eb_input)
    return _tc_tail_call(part, mm_0_a, mm_0_b, W0, W1, W2)


# speedup vs baseline: 1.2024x; 1.0432x over previous
import functools

import jax
import jax.numpy as jnp
from jax import lax
from jax.experimental import pallas as pl
from jax.experimental.pallas import tpu as pltpu
from jax.experimental.pallas import tpu_sc as plsc

L = 16
NW = 16
N_IN = 16384
CPT = N_IN // NW
NV = 5
D = 14

_mesh = plsc.VectorSubcoreMesh(core_axis_name="c", subcore_axis_name="s",
                               num_cores=1)


@functools.partial(
    pl.kernel,
    mesh=_mesh,
    out_type=jax.ShapeDtypeStruct((NW, L), jnp.float32),
    scratch_types=[
        pltpu.VMEM((CPT,), jnp.int32),
        pltpu.VMEM((L,), jnp.float32),
    ],
)
def _sc_histogram(e_hbm, out_hbm, idx_v, part_v):
    s = lax.axis_index("s")
    wid = s
    lane = lax.broadcasted_iota(jnp.int32, (L,), 0)

    def lane_sum(x):
        for sh in (8, 4, 2, 1):
            x = x + x.at[lane ^ sh].get(mode="promise_in_bounds",
                                        unique_indices=True)
        return x

    pltpu.sync_copy(e_hbm.at[pl.ds(wid * CPT, CPT)], idx_v)

    VPI = 4

    def body(it, acc):
        base = it * (VPI * L)
        for k in range(VPI):
            x = jnp.maximum(idx_v[pl.ds(base + k * L, L)], 0)
            acc = tuple(acc[v] + jnp.where(x == v, 1.0, 0.0)
                        for v in range(NV - 1))
        return acc

    acc = lax.fori_loop(0, CPT // (VPI * L), body,
                        tuple(jnp.zeros((L,), jnp.float32)
                              for _ in range(NV - 1)))
    cs = [lane_sum(a) for a in acc]
    cs.append(jnp.float32(CPT) - cs[0] - cs[1] - cs[2] - cs[3])
    part = jnp.zeros((L,), jnp.float32)
    for v in range(NV):
        part = jnp.where(lane == v, cs[v], part)
    part_v[...] = part
    pltpu.sync_copy(part_v, out_hbm.at[wid])


def _tc_tail(part_ref, a_ref, b_ref, w0_ref, w1_ref, w2_ref, out_ref):
    bf = jnp.bfloat16
    counts = jnp.sum(part_ref[...], axis=0)
    c5 = counts[:NV]
    e0 = jnp.sum(jnp.sum(w0_ref[...], axis=0) * c5[:, None], axis=0)
    e1 = jnp.sum(jnp.sum(w1_ref[...], axis=0) * c5[:, None], axis=0)
    e2 = jnp.sum(jnp.sum(w2_ref[...], axis=0) * c5[:, None], axis=0)
    mm0 = jax.lax.dot_general(
        a_ref[...].astype(bf), b_ref[...].astype(bf),
        (((1,), (0,)), ((), ())),
        preferred_element_type=jnp.float32)
    mm1 = mm0 * e0[None, :]
    mm2 = jax.lax.dot_general(
        e1[None, :].astype(bf), mm1.astype(bf),
        (((1,), (0,)), ((), ())),
        preferred_element_type=jnp.float32)
    s = jnp.sum(e2 * mm2[0, :])
    out_ref[...] = jnp.full((1, 1), s, jnp.float32)


_tc_tail_call = pl.pallas_call(
    _tc_tail,
    out_shape=jax.ShapeDtypeStruct((1, 1), jnp.float32),
)


def kernel(mm_0_a, mm_0_b, eb_input, eb_offset, W0, W1, W2):
    del eb_offset
    part = _sc_histogram(eb_input)
    return _tc_tail_call(part, mm_0_a, mm_0_b, W0, W1, W2)


# speedup vs baseline: 1.2132x; 1.0090x over previous
import functools

import jax
import jax.numpy as jnp
from jax import lax
from jax.experimental import pallas as pl
from jax.experimental.pallas import tpu as pltpu
from jax.experimental.pallas import tpu_sc as plsc

L = 16
NW = 16
N_IN = 16384
CPT = N_IN // NW
NV = 5
D = 14

_mesh = plsc.VectorSubcoreMesh(core_axis_name="c", subcore_axis_name="s",
                               num_cores=1)


@functools.partial(
    pl.kernel,
    mesh=_mesh,
    out_type=jax.ShapeDtypeStruct((NW, L), jnp.float32),
    scratch_types=[
        pltpu.VMEM((CPT,), jnp.int32),
        pltpu.VMEM((L,), jnp.float32),
    ],
)
def _sc_histogram(e_hbm, out_hbm, idx_v, part_v):
    s = lax.axis_index("s")
    wid = s
    lane = lax.broadcasted_iota(jnp.int32, (L,), 0)

    def lane_sum(x):
        for sh in (8, 4, 2, 1):
            x = x + x.at[lane ^ sh].get(mode="promise_in_bounds",
                                        unique_indices=True)
        return x

    pltpu.sync_copy(e_hbm.at[pl.ds(wid * CPT, CPT)], idx_v)

    VPI = 2

    def body(it, acc):
        base = it * (VPI * L)
        for k in range(VPI):
            x = jnp.maximum(idx_v[pl.ds(base + k * L, L)], 0)
            acc = tuple(acc[v] + jnp.where(x == v, 1.0, 0.0)
                        for v in range(NV - 1))
        return acc

    acc = lax.fori_loop(0, CPT // (VPI * L), body,
                        tuple(jnp.zeros((L,), jnp.float32)
                              for _ in range(NV - 1)))
    cs = [lane_sum(a) for a in acc]
    cs.append(jnp.float32(CPT) - cs[0] - cs[1] - cs[2] - cs[3])
    part = jnp.zeros((L,), jnp.float32)
    for v in range(NV):
        part = jnp.where(lane == v, cs[v], part)
    part_v[...] = part
    pltpu.sync_copy(part_v, out_hbm.at[wid])


def _tc_tail(part_ref, a_ref, b_ref, w0_ref, w1_ref, w2_ref, out_ref):
    bf = jnp.bfloat16
    counts = jnp.sum(part_ref[...], axis=0)
    c5 = counts[:NV]
    e0 = jnp.sum(jnp.sum(w0_ref[...], axis=0) * c5[:, None], axis=0)
    e1 = jnp.sum(jnp.sum(w1_ref[...], axis=0) * c5[:, None], axis=0)
    e2 = jnp.sum(jnp.sum(w2_ref[...], axis=0) * c5[:, None], axis=0)
    mm0 = jax.lax.dot_general(
        a_ref[...].astype(bf), b_ref[...].astype(bf),
        (((1,), (0,)), ((), ())),
        preferred_element_type=jnp.float32)
    mm1 = mm0 * e0[None, :]
    mm2 = jax.lax.dot_general(
        e1[None, :].astype(bf), mm1.astype(bf),
        (((1,), (0,)), ((), ())),
        preferred_element_type=jnp.float32)
    s = jnp.sum(e2 * mm2[0, :])
    out_ref[...] = jnp.full((1, 1), s, jnp.float32)


_tc_tail_call = pl.pallas_call(
    _tc_tail,
    out_shape=jax.ShapeDtypeStruct((1, 1), jnp.float32),
)


def kernel(mm_0_a, mm_0_b, eb_input, eb_offset, W0, W1, W2):
    del eb_offset
    part = _sc_histogram(eb_input)
    return _tc_tail_call(part, mm_0_a, mm_0_b, W0, W1, W2)
